# baseline (speedup 1.0000x reference)
import functools
import numpy as np
import jax
import jax.numpy as jnp
from jax.experimental import pallas as pl
from jax.experimental.pallas import tpu as pltpu

_K = 4
_EPS = 1e-5
_LANE = 128


@functools.lru_cache(maxsize=None)
def _gcn_constants(V, HW, P):
    N = V * HW
    deg = np.ones(N, np.float64)
    edges = []
    for t in range(V - 1):
        for k in range(_K):
            s, d = t * HW, (t + 1) * HW + k
            edges.append((s, d))
            edges.append((d, s))
    for (_, c) in edges:
        deg[c] += 1.0
    dis = 1.0 / np.sqrt(deg)
    Mfull = np.zeros((V * _K, V * P), np.float64)
    for (r, c) in edges:
        qr = (r // HW) * _K + (r % HW)
        cp = (c // HW) * P + (c % HW)
        Mfull[qr, cp] += dis[r] * dis[c]
    invdeg = np.zeros((1, V * P), np.float64)
    mask = np.zeros((1, V * P), np.float64)
    for n in range(N):
        invdeg[0, (n // HW) * P + (n % HW)] = 1.0 / deg[n]
        mask[0, (n // HW) * P + (n % HW)] = 1.0
    return (np.asarray(invdeg, np.float32), np.asarray(Mfull, np.float32),
            np.asarray(mask, np.float32))


def _shift_add(Z0, Z1, Z2, C, V, P):
    Z0 = Z0.reshape(C, V, P)
    Z2 = Z2.reshape(C, V, P)
    zpad = jnp.zeros((C, 1, P), jnp.float32)
    Y = (Z1.reshape(C, V, P)
         + jnp.concatenate([zpad, Z0[:, :-1, :]], axis=1)
         + jnp.concatenate([Z2[:, 1:, :], zpad], axis=1))
    return Y.reshape(C, V * P)


def _stats_block(Yf, count_blk):
    C = Yf.shape[0]
    s = jnp.sum(Yf, axis=1)[:, None]
    mb = s / count_blk
    d = Yf - mb
    q_all = jnp.sum(d * d, axis=1)[:, None]
    n_pad = Yf.shape[1] - count_blk
    q = q_all - n_pad * mb * mb
    return jnp.concatenate([s, q, jnp.zeros((C, 126), jnp.float32)], axis=1)


def _bn_affine(stats, gamma, beta, count_blk, nblocks):
    C = stats.shape[0]
    st = stats.reshape(C, nblocks, 128)
    s_i = st[:, :, 0]
    q_i = st[:, :, 1]
    total = jnp.sum(s_i, axis=1)[:, None]
    count = count_blk * nblocks
    mean = total / count
    mb = s_i / count_blk
    var = (jnp.sum(q_i, axis=1)[:, None]
           + count_blk * jnp.sum((mb - mean) ** 2, axis=1)[:, None]) / count
    a = gamma / jnp.sqrt(var + _EPS)
    b = beta - mean * a
    return a, b


def _body(x_ref, dw_ref, uw_ref, wt_ref, gb_ref, gdg_ref, gdb_ref,
          gug_ref, gub_ref, inv_ref, m_ref, mask_ref, o_ref,
          y_scr, z_scr, stA_scr, stB_scr, *, Bn, V, HW, P):
    i = pl.program_id(0)
    C = x_ref.shape[1]
    SP = V * P
    f32 = jnp.float32
    cnt = float(V * HW)

    @pl.when(i < Bn)
    def _phase0():
        Xb = x_ref[...]
        Xc = jnp.transpose(Xb, (1, 0, 2))
        Xp = jnp.concatenate(
            [Xc, jnp.zeros((C, V, P - HW), f32)], axis=2).reshape(C, SP)
        w = dw_ref[...]
        Z0 = jnp.dot(w[0], Xp, preferred_element_type=f32)
        Z1 = jnp.dot(w[1], Xp, preferred_element_type=f32)
        Z2 = jnp.dot(w[2], Xp, preferred_element_type=f32)
        Yf = _shift_add(Z0, Z1, Z2, C, V, P)
        y_scr[:, pl.ds(i * SP, SP)] = Yf.astype(jnp.bfloat16)
        stA_scr[:, pl.ds(i * 128, 128)] = _stats_block(Yf, cnt)

    @pl.when((i >= Bn) & (i < 2 * Bn))
    def _phase1():
        j = i - Bn
        Y = y_scr[:, pl.ds(j * SP, SP)].astype(f32)
        mask = mask_ref[...]
        a, b = _bn_affine(stA_scr[...], gdg_ref[...], gdb_ref[...], cnt, Bn)
        Yb = Y * a + b * mask
        XW = jnp.dot(wt_ref[...], Yb, preferred_element_type=f32)
        Xs = XW.reshape(C, V, P)[:, :, :_K].reshape(C, V * _K)
        G = (XW * inv_ref[...]
             + jnp.dot(Xs, m_ref[...], preferred_element_type=f32)
             + gb_ref[...] * mask)
        uw = uw_ref[...]
        Z0 = jnp.dot(uw[0], G, preferred_element_type=f32)
        Z1 = jnp.dot(uw[1], G, preferred_element_type=f32)
        Z2 = jnp.dot(uw[2], G, preferred_element_type=f32)
        Zf = _shift_add(Z0, Z1, Z2, C, V, P)
        z_scr[:, pl.ds(j * SP, SP)] = Zf.astype(jnp.bfloat16)
        stB_scr[:, pl.ds(j * 128, 128)] = _stats_block(Zf, cnt)

    @pl.when(i >= 2 * Bn)
    def _phase2():
        j = i - 2 * Bn
        Z = z_scr[:, pl.ds(j * SP, SP)].astype(f32)
        a, b = _bn_affine(stB_scr[...], gug_ref[...], gub_ref[...], cnt, Bn)
        Zn = (Z * a + b).reshape(C, V, P)[:, :, :HW]
        o_ref[...] = jnp.transpose(Zn, (1, 0, 2))


def kernel(x, batch, down_w, down_gamma, down_beta, up_w, up_gamma, up_beta,
           gcn_w, gcn_b):
    tlen, C, H, W = x.shape
    try:
        Bn = int(batch)
    except Exception:
        Bn = 4
    V = tlen // Bn
    HW = H * W
    P = -(-HW // _LANE) * _LANE
    SP = V * P
    VK = V * _K

    invdeg_np, Mfull_np, mask_np = _gcn_constants(V, HW, P)
    invdeg = jnp.asarray(invdeg_np)
    Mfull = jnp.asarray(Mfull_np)
    mask = jnp.asarray(mask_np)

    dw3 = jnp.transpose(down_w.reshape(C, C, 3), (2, 0, 1))
    uw3 = jnp.transpose(up_w.reshape(C, C, 3), (2, 0, 1))
    xr = x.reshape(tlen, C, HW)

    parked = lambda i: (0, 0)
    parked3 = lambda i: (0, 0, 0)
    zo = pl.pallas_call(
        functools.partial(_body, Bn=Bn, V=V, HW=HW, P=P),
        grid=(3 * Bn,),
        in_specs=[
            pl.BlockSpec((V, C, HW), lambda i: (jnp.minimum(i, Bn - 1), 0, 0)),
            pl.BlockSpec((3, C, C), parked3),
            pl.BlockSpec((3, C, C), parked3),
            pl.BlockSpec((C, C), parked),
            pl.BlockSpec((C, 1), parked),
            pl.BlockSpec((C, 1), parked),
            pl.BlockSpec((C, 1), parked),
            pl.BlockSpec((C, 1), parked),
            pl.BlockSpec((C, 1), parked),
            pl.BlockSpec((1, SP), parked),
            pl.BlockSpec((VK, SP), parked),
            pl.BlockSpec((1, SP), parked),
        ],
        out_specs=pl.BlockSpec(
            (V, C, HW), lambda i: (jnp.maximum(i - 2 * Bn + 1, 0), 0, 0)),
        out_shape=jax.ShapeDtypeStruct(((Bn + 1) * V, C, HW), jnp.float32),
        scratch_shapes=[
            pltpu.VMEM((C, Bn * SP), jnp.bfloat16),
            pltpu.VMEM((C, Bn * SP), jnp.bfloat16),
            pltpu.VMEM((C, Bn * 128), jnp.float32),
            pltpu.VMEM((C, Bn * 128), jnp.float32),
        ],
        compiler_params=pltpu.CompilerParams(
            dimension_semantics=("arbitrary",)),
    )(xr, dw3, uw3, jnp.transpose(gcn_w), gcn_b.reshape(C, 1),
      down_gamma.reshape(C, 1), down_beta.reshape(C, 1),
      up_gamma.reshape(C, 1), up_beta.reshape(C, 1), invdeg, Mfull, mask)

    return zo[V:].reshape(tlen, C, H, W)


# speedup vs baseline: 1.0397x; 1.0397x over previous
import functools
import numpy as np
import jax
import jax.numpy as jnp
from jax.experimental import pallas as pl

_K = 4
_EPS = 1e-5
_LANE = 128


@functools.lru_cache(maxsize=None)
def _gcn_constants(V, HW, P):
    N = V * HW
    deg = np.ones(N, np.float64)
    edges = []
    for t in range(V - 1):
        for k in range(_K):
            s, d = t * HW, (t + 1) * HW + k
            edges.append((s, d))
            edges.append((d, s))
    for (_, c) in edges:
        deg[c] += 1.0
    dis = 1.0 / np.sqrt(deg)
    Mfull = np.zeros((V * _K, V * P), np.float64)
    for (r, c) in edges:
        qr = (r // HW) * _K + (r % HW)
        cp = (c // HW) * P + (c % HW)
        Mfull[qr, cp] += dis[r] * dis[c]
    invdeg = np.zeros((1, V * P), np.float64)
    for n in range(N):
        invdeg[0, (n // HW) * P + (n % HW)] = 1.0 / deg[n]
    mask = np.zeros((1, V * P), np.float64)
    for n in range(N):
        mask[0, (n // HW) * P + (n % HW)] = 1.0
    return (np.asarray(invdeg, np.float32), np.asarray(Mfull, np.float32),
            np.asarray(mask, np.float32))


def _shift_add(Z0, Z1, Z2, C, V, P):
    Z0 = Z0.reshape(C, V, P)
    Z2 = Z2.reshape(C, V, P)
    zpad = jnp.zeros((C, 1, P), jnp.float32)
    Y = (Z1.reshape(C, V, P)
         + jnp.concatenate([zpad, Z0[:, :-1, :]], axis=1)
         + jnp.concatenate([Z2[:, 1:, :], zpad], axis=1))
    return Y.reshape(C, V * P)


def _stats_block(Yf, count_blk):
    C = Yf.shape[0]
    s = jnp.sum(Yf, axis=1)[:, None]
    mb = s / count_blk
    d = Yf - mb
    q_all = jnp.sum(d * d, axis=1)[:, None]
    n_pad = Yf.shape[1] - count_blk
    q = q_all - n_pad * mb * mb
    return jnp.concatenate([s, q, jnp.zeros((C, 126), jnp.float32)], axis=1)


def _bn_affine(stats, gamma, beta, count_blk, nblocks):
    C = stats.shape[0]
    st = stats.reshape(C, nblocks, 128)
    s_i = st[:, :, 0]
    q_i = st[:, :, 1]
    total = jnp.sum(s_i, axis=1)[:, None]
    count = count_blk * nblocks
    mean = total / count
    mb = s_i / count_blk
    var = (jnp.sum(q_i, axis=1)[:, None]
           + count_blk * jnp.sum((mb - mean) ** 2, axis=1)[:, None]) / count
    a = gamma / jnp.sqrt(var + _EPS)
    b = beta - mean * a
    return a, b


def _ka_body(x_ref, w_ref, y_ref, st_ref, *, V, HW, P):
    Xb = x_ref[...]
    C = Xb.shape[1]
    Xc = jnp.transpose(Xb, (1, 0, 2))
    Xp = jnp.concatenate(
        [Xc, jnp.zeros((C, V, P - HW), jnp.float32)], axis=2).reshape(C, V * P)
    w = w_ref[...]
    f32 = jnp.float32
    Z0 = jnp.dot(w[0], Xp, preferred_element_type=f32)
    Z1 = jnp.dot(w[1], Xp, preferred_element_type=f32)
    Z2 = jnp.dot(w[2], Xp, preferred_element_type=f32)
    Yf = _shift_add(Z0, Z1, Z2, C, V, P)
    y_ref[...] = Yf
    st_ref[...] = _stats_block(Yf, float(V * HW))


def _kb_body(y_ref, stA_ref, g_ref, bta_ref, wt_ref, gb_ref, inv_ref, m_ref,
             mask_ref, uw_ref, z_ref, stB_ref, *, V, HW, P, Bn):
    Y = y_ref[...]
    C = Y.shape[0]
    mask = mask_ref[...]
    a, b = _bn_affine(stA_ref[...], g_ref[...], bta_ref[...],
                      float(V * HW), Bn)
    Yb = Y * a + b * mask
    XW = jnp.dot(wt_ref[...], Yb, preferred_element_type=jnp.float32)
    Xs = XW.reshape(C, V, P)[:, :, :_K].reshape(C, V * _K)
    G = (XW * inv_ref[...]
         + jnp.dot(Xs, m_ref[...], preferred_element_type=jnp.float32)
         + gb_ref[...] * mask)
    uw = uw_ref[...]
    f32 = jnp.float32
    Z0 = jnp.dot(uw[0], G, preferred_element_type=f32)
    Z1 = jnp.dot(uw[1], G, preferred_element_type=f32)
    Z2 = jnp.dot(uw[2], G, preferred_element_type=f32)
    Zf = _shift_add(Z0, Z1, Z2, C, V, P)
    z_ref[...] = Zf
    stB_ref[...] = _stats_block(Zf, float(V * HW))


def _kc_body(z_ref, stB_ref, g_ref, bta_ref, o_ref, *, V, HW, P, Bn):
    Z = z_ref[...]
    C = Z.shape[0]
    a, b = _bn_affine(stB_ref[...], g_ref[...], bta_ref[...],
                      float(V * HW), Bn)
    Zn = (Z * a + b).reshape(C, V, P)[:, :, :HW]
    o_ref[...] = jnp.transpose(Zn, (1, 0, 2))


def kernel(x, batch, down_w, down_gamma, down_beta, up_w, up_gamma, up_beta,
           gcn_w, gcn_b):
    tlen, C, H, W = x.shape
    try:
        Bn = int(batch)
    except Exception:
        Bn = 4
    V = tlen // Bn
    HW = H * W
    P = -(-HW // _LANE) * _LANE
    SP = V * P
    VK = V * _K

    invdeg_np, Mfull_np, mask_np = _gcn_constants(V, HW, P)
    invdeg = jnp.asarray(invdeg_np)
    Mfull = jnp.asarray(Mfull_np)
    mask = jnp.asarray(mask_np)

    dw3 = jnp.transpose(down_w.reshape(C, C, 3), (2, 0, 1))
    uw3 = jnp.transpose(up_w.reshape(C, C, 3), (2, 0, 1))
    xr = x.reshape(tlen, C, HW)

    y_raw, stA = pl.pallas_call(
        functools.partial(_ka_body, V=V, HW=HW, P=P),
        grid=(Bn,),
        in_specs=[
            pl.BlockSpec((V, C, HW), lambda i: (i, 0, 0)),
            pl.BlockSpec((3, C, C), lambda i: (0, 0, 0)),
        ],
        out_specs=[
            pl.BlockSpec((C, SP), lambda i: (0, i)),
            pl.BlockSpec((C, 128), lambda i: (0, i)),
        ],
        out_shape=[
            jax.ShapeDtypeStruct((C, Bn * SP), jnp.float32),
            jax.ShapeDtypeStruct((C, Bn * 128), jnp.float32),
        ],
    )(xr, dw3)

    z_raw, stB = pl.pallas_call(
        functools.partial(_kb_body, V=V, HW=HW, P=P, Bn=Bn),
        grid=(Bn,),
        in_specs=[
            pl.BlockSpec((C, SP), lambda i: (0, i)),
            pl.BlockSpec((C, Bn * 128), lambda i: (0, 0)),
            pl.BlockSpec((C, 1), lambda i: (0, 0)),
            pl.BlockSpec((C, 1), lambda i: (0, 0)),
            pl.BlockSpec((C, C), lambda i: (0, 0)),
            pl.BlockSpec((C, 1), lambda i: (0, 0)),
            pl.BlockSpec((1, SP), lambda i: (0, 0)),
            pl.BlockSpec((VK, SP), lambda i: (0, 0)),
            pl.BlockSpec((1, SP), lambda i: (0, 0)),
            pl.BlockSpec((3, C, C), lambda i: (0, 0, 0)),
        ],
        out_specs=[
            pl.BlockSpec((C, SP), lambda i: (0, i)),
            pl.BlockSpec((C, 128), lambda i: (0, i)),
        ],
        out_shape=[
            jax.ShapeDtypeStruct((C, Bn * SP), jnp.float32),
            jax.ShapeDtypeStruct((C, Bn * 128), jnp.float32),
        ],
    )(y_raw, stA, down_gamma.reshape(C, 1), down_beta.reshape(C, 1),
      jnp.transpose(gcn_w), gcn_b.reshape(C, 1), invdeg, Mfull, mask, uw3)

    z = pl.pallas_call(
        functools.partial(_kc_body, V=V, HW=HW, P=P, Bn=Bn),
        grid=(Bn,),
        in_specs=[
            pl.BlockSpec((C, SP), lambda i: (0, i)),
            pl.BlockSpec((C, Bn * 128), lambda i: (0, 0)),
            pl.BlockSpec((C, 1), lambda i: (0, 0)),
            pl.BlockSpec((C, 1), lambda i: (0, 0)),
        ],
        out_specs=pl.BlockSpec((V, C, HW), lambda i: (i, 0, 0)),
        out_shape=jax.ShapeDtypeStruct((tlen, C, HW), jnp.float32),
    )(z_raw, stB, up_gamma.reshape(C, 1), up_beta.reshape(C, 1))

    return z.reshape(tlen, C, H, W)
